# baseline (speedup 1.0000x reference)
import functools

import jax
import jax.numpy as jnp
import numpy as np
from jax import lax
from jax.experimental import pallas as pl
from jax.experimental.pallas import tpu as pltpu
from jax.experimental.pallas import tpu_sc as plsc

B, A, C = 16, 65536, 16
N = B * A
NC, NS, L = 2, 16, 16
NW = NC * NS
ROWS_PER_W = N // NW
CHUNK = 2048
NCHUNK = ROWS_PER_W // CHUNK
GROUPS = CHUNK // L

_LN2 = float(np.log(2.0))


def _log_from_bits(s):
    bits = plsc.bitcast(s, jnp.int32)
    e = (bits >> 23) - 127
    mant = plsc.bitcast((bits & 0x007FFFFF) | 0x3F800000, jnp.float32)
    t = (mant - 1.0) / (mant + 1.0)
    t2 = t * t
    p = jnp.full((L,), 1.0 / 9.0, dtype=jnp.float32)
    for coef in (1.0 / 7.0, 1.0 / 5.0, 1.0 / 3.0, 1.0):
        p = p * t2 + coef
    return _LN2 * e.astype(jnp.float32) + 2.0 * t * p


def _sc_body(logits_hbm, labels_hbm, tags_hbm, acc_hbm, cnt_hbm,
             logits_v, labels_v, tags_v, out_v):
    wid = lax.axis_index("c") * NS + lax.axis_index("s")
    row0 = wid * ROWS_PER_W
    iota = lax.iota(jnp.int32, L)
    iota_c = iota * C

    def chunk_body(j, carry):
        acc, cnt = carry
        r0 = row0 + j * CHUNK
        pltpu.sync_copy(logits_hbm.at[pl.ds(r0 * C, CHUNK * C)], logits_v)
        pltpu.sync_copy(labels_hbm.at[pl.ds(r0, CHUNK)], labels_v)
        pltpu.sync_copy(tags_hbm.at[pl.ds(r0, CHUNK)], tags_v)

        def group_body(g, carry):
            acc, cnt = carry
            base = g * (L * C)
            cols = [
                plsc.load_gather(logits_v, [base + iota_c + c])
                for c in range(C)
            ]
            m = cols[0]
            for c in range(1, C):
                m = jnp.maximum(m, cols[c])
            s = jnp.exp(cols[0] - m)
            for c in range(1, C):
                s = s + jnp.exp(cols[c] - m)
            lse = m + _log_from_bits(s)
            lab = labels_v[pl.ds(g * L, L)]
            vlab = plsc.load_gather(logits_v, [base + iota_c + lab])
            tag = tags_v[pl.ds(g * L, L)]
            msk = tag != 0.0
            acc = acc + jnp.where(msk, lse - vlab, 0.0)
            cnt = cnt + jnp.where(msk, 1.0, 0.0)
            return acc, cnt

        return lax.fori_loop(0, GROUPS, group_body, (acc, cnt))

    zero = jnp.zeros((L,), jnp.float32)
    acc, cnt = lax.fori_loop(0, NCHUNK, chunk_body, (zero, zero))
    out_v[...] = acc
    pltpu.sync_copy(out_v, acc_hbm.at[wid])
    out_v[...] = cnt
    pltpu.sync_copy(out_v, cnt_hbm.at[wid])


@functools.partial(
    pl.kernel,
    out_type=(
        jax.ShapeDtypeStruct((NW, L), jnp.float32),
        jax.ShapeDtypeStruct((NW, L), jnp.float32),
    ),
    mesh=plsc.VectorSubcoreMesh(
        core_axis_name="c", subcore_axis_name="s", num_cores=NC,
        num_subcores=NS,
    ),
    scratch_types=[
        pltpu.VMEM((CHUNK * C,), jnp.float32),
        pltpu.VMEM((CHUNK,), jnp.int32),
        pltpu.VMEM((CHUNK,), jnp.float32),
        pltpu.VMEM((L,), jnp.float32),
    ],
    compiler_params=pltpu.CompilerParams(needs_layout_passes=False),
)
def _sc_partials(*args):
    _sc_body(*args)


def _finish_body(acc_ref, cnt_ref, out_ref):
    total = jnp.sum(acc_ref[...])
    count = jnp.sum(cnt_ref[...])
    out_ref[0, 0] = total / jnp.maximum(count, 1.0)


_finish = pl.pallas_call(
    _finish_body,
    out_shape=jax.ShapeDtypeStruct((1, 1), jnp.float32),
    out_specs=pl.BlockSpec(memory_space=pltpu.SMEM),
)


def kernel(predict_cls_logits, true_cls_ids, anchors_tag):
    logits = predict_cls_logits.reshape(N * C)
    labels = true_cls_ids.reshape(N)
    tags = anchors_tag.reshape(N)
    acc, cnt = _sc_partials(logits, labels, tags)
    return _finish(acc, cnt)[0, 0]


# speedup vs baseline: 1.0774x; 1.0774x over previous
import functools

import jax
import jax.numpy as jnp
import numpy as np
from jax import lax
from jax.experimental import pallas as pl
from jax.experimental.pallas import tpu as pltpu
from jax.experimental.pallas import tpu_sc as plsc

B, A, C = 16, 65536, 16
N = B * A
NC, NS, L = 2, 16, 16
NW = NC * NS
ROWS_PER_W = N // NW
CHUNK = 2048
NCHUNK = ROWS_PER_W // CHUNK
GROUPS = CHUNK // L
UNROLL = 2

_LN2 = float(np.log(2.0))

_xs = np.linspace(1.0, 2.0, 8193)
_LOG2_COEFS = tuple(
    float(c)
    for c in np.polynomial.chebyshev.Chebyshev.fit(_xs, np.log2(_xs), 6)
    .convert(kind=np.polynomial.Polynomial)
    .coef
)


def _log_from_bits(s):
    bits = plsc.bitcast(s, jnp.int32)
    e = (bits >> 23) - 127
    mant = plsc.bitcast((bits & 0x007FFFFF) | 0x3F800000, jnp.float32)
    p = jnp.full((L,), _LOG2_COEFS[-1], dtype=jnp.float32)
    for coef in _LOG2_COEFS[-2::-1]:
        p = p * mant + coef
    return _LN2 * (e.astype(jnp.float32) + p)


def _tree(op, xs):
    while len(xs) > 1:
        nxt = [op(xs[i], xs[i + 1]) for i in range(0, len(xs) - 1, 2)]
        if len(xs) % 2:
            nxt.append(xs[-1])
        xs = nxt
    return xs[0]


def _sc_body(logits_hbm, labels_hbm, tags_hbm, acc_hbm, cnt_hbm,
             logits_b0, logits_b1, labels_b0, labels_b1, tags_b0, tags_b1,
             out_v, sem0, sem1):
    wid = lax.axis_index("c") * NS + lax.axis_index("s")
    row0 = wid * ROWS_PER_W
    iota = lax.iota(jnp.int32, L)
    iota_c = iota * C

    bufs = ((logits_b0, labels_b0, tags_b0, sem0),
            (logits_b1, labels_b1, tags_b1, sem1))

    def issue(j):
        lg, lb, tg, sem = bufs[j % 2]
        r0 = row0 + j * CHUNK
        return (
            pltpu.async_copy(logits_hbm.at[pl.ds(r0 * C, CHUNK * C)], lg, sem),
            pltpu.async_copy(labels_hbm.at[pl.ds(r0, CHUNK)], lb, sem),
            pltpu.async_copy(tags_hbm.at[pl.ds(r0, CHUNK)], tg, sem),
        )

    def do_group(g, buf, acc, cnt):
        lg, lb, tg, _ = buf
        ebase = g * (L * C)
        idx0 = ebase + iota_c
        cols = [plsc.load_gather(lg, [idx0 if c == 0 else idx0 + c])
                for c in range(C)]
        m = _tree(jnp.maximum, cols)
        s = _tree(jnp.add, [jnp.exp(col - m) for col in cols])
        lse = m + _log_from_bits(s)
        lab = lb[pl.ds(g * L, L)]
        vlab = plsc.load_gather(lg, [idx0 + lab])
        tag = tg[pl.ds(g * L, L)]
        msk = tag != 0.0
        acc = acc + jnp.where(msk, lse - vlab, 0.0)
        cnt = cnt + jnp.where(msk, 1.0, 0.0)
        return acc, cnt

    zero = jnp.zeros((L,), jnp.float32)
    accs = [zero] * UNROLL
    cnts = [zero] * UNROLL

    pending = [issue(0), issue(1)]
    for j in range(NCHUNK):
        buf = bufs[j % 2]
        for d in pending[j % 2]:
            d.wait()
        if j + 2 < NCHUNK:
            pending[j % 2] = issue(j + 2)

        def chunk_loop(h, carry, _buf=buf):
            res = []
            for u in range(UNROLL):
                a, c = do_group(h * UNROLL + u, _buf,
                                carry[2 * u], carry[2 * u + 1])
                res += [a, c]
            return tuple(res)

        flat = []
        for u in range(UNROLL):
            flat += [accs[u], cnts[u]]
        flat = lax.fori_loop(0, GROUPS // UNROLL, chunk_loop, tuple(flat))
        accs = [flat[2 * u] for u in range(UNROLL)]
        cnts = [flat[2 * u + 1] for u in range(UNROLL)]

    acc = _tree(jnp.add, accs)
    cnt = _tree(jnp.add, cnts)
    out_v[...] = acc
    pltpu.sync_copy(out_v, acc_hbm.at[wid])
    out_v[...] = cnt
    pltpu.sync_copy(out_v, cnt_hbm.at[wid])


@functools.partial(
    pl.kernel,
    out_type=(
        jax.ShapeDtypeStruct((NW, L), jnp.float32),
        jax.ShapeDtypeStruct((NW, L), jnp.float32),
    ),
    mesh=plsc.VectorSubcoreMesh(
        core_axis_name="c", subcore_axis_name="s", num_cores=NC,
        num_subcores=NS,
    ),
    scratch_types=[
        pltpu.VMEM((CHUNK * C,), jnp.float32),
        pltpu.VMEM((CHUNK * C,), jnp.float32),
        pltpu.VMEM((CHUNK,), jnp.int32),
        pltpu.VMEM((CHUNK,), jnp.int32),
        pltpu.VMEM((CHUNK,), jnp.float32),
        pltpu.VMEM((CHUNK,), jnp.float32),
        pltpu.VMEM((L,), jnp.float32),
        pltpu.SemaphoreType.DMA,
        pltpu.SemaphoreType.DMA,
    ],
    compiler_params=pltpu.CompilerParams(needs_layout_passes=False),
)
def _sc_partials(*args):
    _sc_body(*args)


def _finish_body(acc_ref, cnt_ref, out_ref):
    total = jnp.sum(acc_ref[...])
    count = jnp.sum(cnt_ref[...])
    out_ref[0, 0] = total / jnp.maximum(count, 1.0)


_finish = pl.pallas_call(
    _finish_body,
    out_shape=jax.ShapeDtypeStruct((1, 1), jnp.float32),
    out_specs=pl.BlockSpec(memory_space=pltpu.SMEM),
)


def kernel(predict_cls_logits, true_cls_ids, anchors_tag):
    logits = predict_cls_logits.reshape(N * C)
    labels = true_cls_ids.reshape(N)
    tags = anchors_tag.reshape(N)
    acc, cnt = _sc_partials(logits, labels, tags)
    return _finish(acc, cnt)[0, 0]
